# baseline (speedup 1.0000x reference)
import functools

import jax
from jax import lax
from jax.experimental import pallas as pl
from jax.experimental.pallas import tpu as pltpu
from jax.experimental.pallas import tpu_sc as plsc

_N_CHUNKS = 1


def kernel(x, pe):
    seq_len = x.shape[1]
    d_model = pe.shape[2]
    info = plsc.get_sparse_core_info()
    nw = info.num_cores * info.num_subcores
    rows_per_w = seq_len // nw
    chunk_rows = rows_per_w // _N_CHUNKS
    mesh = plsc.VectorSubcoreMesh(core_axis_name="c", subcore_axis_name="s")

    @functools.partial(
        pl.kernel,
        mesh=mesh,
        out_type=jax.ShapeDtypeStruct((1, seq_len, d_model), pe.dtype),
        scratch_types=[
            pltpu.VMEM((_N_CHUNKS, chunk_rows, d_model), pe.dtype),
            pltpu.SemaphoreType.DMA((_N_CHUNKS,)),
            pltpu.SemaphoreType.DMA((_N_CHUNKS,)),
        ],
    )
    def sc_copy(pe_hbm, out_hbm, buf, in_sems, out_sems):
        wid = lax.axis_index("s") * info.num_cores + lax.axis_index("c")
        base = wid * rows_per_w

        def cp_in(i):
            return pltpu.make_async_copy(
                pe_hbm.at[0, pl.ds(base + i * chunk_rows, chunk_rows), :],
                buf.at[i],
                in_sems.at[i],
            )

        def cp_out(i):
            return pltpu.make_async_copy(
                buf.at[i],
                out_hbm.at[0, pl.ds(base + i * chunk_rows, chunk_rows), :],
                out_sems.at[i],
            )

        for i in range(_N_CHUNKS):
            cp_in(i).start()
        for i in range(_N_CHUNKS):
            cp_in(i).wait()
            cp_out(i).start()
        for i in range(_N_CHUNKS):
            cp_out(i).wait()

    return sc_copy(pe)


# speedup vs baseline: 4.2027x; 4.2027x over previous
import jax
from jax.experimental import pallas as pl
from jax.experimental.pallas import tpu as pltpu


def _make_body(n_chunks, chunk_rows):
    def body(pe_ref, out_ref, buf, in_sems, out_sems):
        def cp_in(i):
            return pltpu.make_async_copy(
                pe_ref.at[:, pl.ds(i * chunk_rows, chunk_rows), :],
                buf.at[i],
                in_sems.at[i],
            )

        def cp_out(i):
            return pltpu.make_async_copy(
                buf.at[i],
                out_ref.at[:, pl.ds(i * chunk_rows, chunk_rows), :],
                out_sems.at[i],
            )

        for i in range(n_chunks):
            cp_in(i).start()
        for i in range(n_chunks):
            cp_in(i).wait()
            cp_out(i).start()
        for i in range(n_chunks):
            cp_out(i).wait()

    return body


def kernel(x, pe):
    seq_len = x.shape[1]
    d_model = pe.shape[2]
    n_chunks = 2 if seq_len % 2 == 0 and seq_len >= 2 else 1
    chunk_rows = seq_len // n_chunks
    out_shape = jax.ShapeDtypeStruct((1, seq_len, d_model), pe.dtype)
    return pl.pallas_call(
        _make_body(n_chunks, chunk_rows),
        out_shape=out_shape,
        in_specs=[pl.BlockSpec(memory_space=pl.ANY)],
        out_specs=pl.BlockSpec(memory_space=pl.ANY),
        scratch_shapes=[
            pltpu.VMEM((n_chunks, 1, chunk_rows, d_model), pe.dtype),
            pltpu.SemaphoreType.DMA((n_chunks,)),
            pltpu.SemaphoreType.DMA((n_chunks,)),
        ],
    )(pe)
